# baseline (speedup 1.0000x reference)
import functools

import jax
import jax.numpy as jnp
from jax import lax
from jax.experimental import pallas as pl
from jax.experimental.pallas import tpu as pltpu
from jax.experimental.pallas import tpu_sc as plsc

D_V = 1_000_000
D_M = 64
B = 16384
CPR = D_V // 16

NC = 2
NS = 16
NW = NC * NS
ROWS_PER_W = D_M // NW
NROW = 16
NCOL = B // NROW


def _body(cidx_hbm, table_hbm, out_hbm, cidx_v, cbuf, row_buf, sem):
    cid = lax.axis_index("c")
    sid = lax.axis_index("s")
    wid = sid * NC + cid

    for rr in range(ROWS_PER_W):
        r = wid * ROWS_PER_W + rr
        pltpu.sync_copy(cidx_hbm.at[r], cidx_v.at[rr])

    for rr in range(ROWS_PER_W):
        r = wid * ROWS_PER_W + rr
        pltpu.sync_copy(row_buf, out_hbm.at[r])


def kernel(token_indices, lookup):
    idx = token_indices.astype(jnp.int32)
    cidx = (idx[None, :] >> 4) + (
        jnp.arange(D_M, dtype=jnp.int32)[:, None] * CPR
    )
    cidx3 = cidx.reshape(D_M, NROW, NCOL)
    mesh = plsc.VectorSubcoreMesh(core_axis_name="c", subcore_axis_name="s")
    k = functools.partial(
        pl.kernel,
        mesh=mesh,
        out_type=jax.ShapeDtypeStruct((D_M, NROW, NCOL), jnp.float32),
        scratch_types=[
            pltpu.VMEM((ROWS_PER_W, NROW, NCOL), jnp.int32),
            pltpu.VMEM((2, NCOL, 16), jnp.float32),
            pltpu.VMEM((NROW, NCOL), jnp.float32),
            pltpu.SemaphoreType.DMA,
        ],
        compiler_params=pltpu.CompilerParams(use_tc_tiling_on_sc=False),
    )(_body)
    out3 = k(cidx3, lookup)
    return out3.reshape(D_M, B)
